# Initial kernel scaffold; baseline (speedup 1.0000x reference)
import jax
import jax.numpy as jnp
from jax.experimental import pallas as pl


def kernel(phoneme_ids, embedding_table):
    raise NotImplementedError("write your pallas kernel here")


# speedup vs baseline: 1.1110x; 1.1110x over previous
import functools

import jax
import jax.numpy as jnp
from jax import lax
from jax.experimental import pallas as pl
from jax.experimental.pallas import tpu as pltpu
from jax.experimental.pallas import tpu_sc as plsc

NUM_ROWS = 1_000_000
DIM = 32
BATCH = 16384 * 50

NC = 2
NS = 16
NW = NC * NS
B_PER_W = BATCH // NW
CHUNK = 3200
N_CHUNKS = B_PER_W // CHUNK

_mesh = plsc.VectorSubcoreMesh(core_axis_name="c", subcore_axis_name="s")


@functools.partial(
    pl.kernel,
    mesh=_mesh,
    out_type=jax.ShapeDtypeStruct((BATCH, DIM), jnp.float32),
    compiler_params=pltpu.CompilerParams(use_tc_tiling_on_sc=False),
    scratch_types=[
        pltpu.VMEM((CHUNK,), jnp.int32),
        pltpu.VMEM((CHUNK, DIM), jnp.float32),
        pltpu.SemaphoreType.DMA,
    ],
)
def _gather_kernel(table_hbm, idx_hbm, out_hbm, idx_v, rows_v, sem):
    wid = lax.axis_index("s") * NC + lax.axis_index("c")
    base = wid * B_PER_W

    def body(i, carry):
        off = base + i * CHUNK
        pltpu.sync_copy(idx_hbm.at[pl.ds(off, CHUNK)], idx_v)
        pltpu.async_copy(table_hbm.at[idx_v], rows_v, sem).wait()
        pltpu.sync_copy(rows_v, out_hbm.at[pl.ds(off, CHUNK)])
        return carry

    lax.fori_loop(0, N_CHUNKS, body, 0)


def kernel(phoneme_ids, embedding_table):
    idx = phoneme_ids.reshape(-1).astype(jnp.int32)
    out = _gather_kernel(embedding_table, idx)
    return out.reshape(phoneme_ids.shape + (DIM,))


# speedup vs baseline: 1.1128x; 1.0016x over previous
import functools

import jax
import jax.numpy as jnp
from jax import lax
from jax.experimental import pallas as pl
from jax.experimental.pallas import tpu as pltpu
from jax.experimental.pallas import tpu_sc as plsc

NUM_ROWS = 1_000_000
DIM = 32
BATCH = 16384 * 50

NC = 2
NS = 16
NW = NC * NS
B_PER_W = BATCH // NW
CHUNK = 800
NBUF = 4
N_CHUNKS = B_PER_W // CHUNK
N_OUTER = N_CHUNKS // NBUF

_mesh = plsc.VectorSubcoreMesh(core_axis_name="c", subcore_axis_name="s")


@functools.partial(
    pl.kernel,
    mesh=_mesh,
    out_type=jax.ShapeDtypeStruct((BATCH, DIM), jnp.float32),
    compiler_params=pltpu.CompilerParams(use_tc_tiling_on_sc=False),
    scratch_types=[
        pltpu.VMEM((B_PER_W,), jnp.int32),
        [pltpu.VMEM((CHUNK, DIM), jnp.float32) for _ in range(NBUF)],
        [pltpu.SemaphoreType.DMA for _ in range(NBUF)],
        [pltpu.SemaphoreType.DMA for _ in range(NBUF)],
    ],
)
def _gather_kernel(table_hbm, idx_hbm, out_hbm, idx_v, rows, sem_g, sem_o):
    wid = lax.axis_index("s") * NC + lax.axis_index("c")
    base = wid * B_PER_W

    pltpu.sync_copy(idx_hbm.at[pl.ds(base, B_PER_W)], idx_v)

    def gather_desc(g, b):
        return pltpu.make_async_copy(
            table_hbm.at[idx_v.at[pl.ds(g * CHUNK, CHUNK)]], rows[b], sem_g[b]
        )

    def store_desc(g, b):
        return pltpu.make_async_copy(
            rows[b], out_hbm.at[pl.ds(base + g * CHUNK, CHUNK)], sem_o[b]
        )

    for b in range(NBUF):
        gather_desc(b, b).start()

    def body(outer, carry):
        for b in range(NBUF):
            g = outer * NBUF + b
            gather_desc(g, b).wait()
            st = store_desc(g, b)
            st.start()
            st.wait()
            gather_desc(g + NBUF, b).start()
        return carry

    lax.fori_loop(0, N_OUTER - 1, body, 0)

    last = []
    for b in range(NBUF):
        g = N_CHUNKS - NBUF + b
        gather_desc(g, b).wait()
        st = store_desc(g, b)
        st.start()
        last.append(st)
    for st in last:
        st.wait()


def kernel(phoneme_ids, embedding_table):
    idx = phoneme_ids.reshape(-1).astype(jnp.int32)
    out = _gather_kernel(embedding_table, idx)
    return out.reshape(phoneme_ids.shape + (DIM,))


# speedup vs baseline: 1.8103x; 1.6267x over previous
import functools

import jax
import jax.numpy as jnp
from jax import lax
from jax.experimental import pallas as pl
from jax.experimental.pallas import tpu as pltpu
from jax.experimental.pallas import tpu_sc as plsc

NUM_ROWS = 1_000_000
DIM = 32
IDS = 16384
SEQ = 50
BATCH = IDS * SEQ

NC = 2
NS = 16
NW = NC * NS
B_PER_W = BATCH // NW
CHUNK = 800
PAGES = CHUNK // SEQ
NBUF = 4
N_CHUNKS = B_PER_W // CHUNK
N_OUTER = N_CHUNKS // NBUF

_mesh = plsc.VectorSubcoreMesh(core_axis_name="c", subcore_axis_name="s")


@functools.partial(
    pl.kernel,
    mesh=_mesh,
    out_type=jax.ShapeDtypeStruct((IDS, SEQ, DIM), jnp.float32),
    compiler_params=pltpu.CompilerParams(use_tc_tiling_on_sc=False),
    scratch_types=[
        pltpu.VMEM((B_PER_W,), jnp.int32),
        [pltpu.VMEM((CHUNK, DIM), jnp.float32) for _ in range(NBUF)],
        [pltpu.SemaphoreType.DMA for _ in range(NBUF)],
        [pltpu.SemaphoreType.DMA for _ in range(NBUF)],
    ],
)
def _gather_kernel(table_hbm, idx_hbm, out_hbm, idx_v, rows, sem_g, sem_o):
    wid = lax.axis_index("s") * NC + lax.axis_index("c")
    base = wid * B_PER_W

    pltpu.sync_copy(idx_hbm.at[pl.ds(base, B_PER_W)], idx_v)

    def gather_desc(g, b):
        return pltpu.make_async_copy(
            table_hbm.at[idx_v.at[pl.ds(g * CHUNK, CHUNK)]], rows[b], sem_g[b]
        )

    def store_descs(g, b):
        page0 = (base + g * CHUNK) // SEQ
        return [
            pltpu.make_async_copy(
                rows[b].at[pl.ds(p * SEQ, SEQ)], out_hbm.at[page0 + p], sem_o[b]
            )
            for p in range(PAGES)
        ]

    for b in range(NBUF):
        gather_desc(b, b).start()

    def body(outer, carry):
        for b in range(NBUF):
            g = outer * NBUF + b
            gather_desc(g, b).wait()
            sts = store_descs(g, b)
            for st in sts:
                st.start()
            for st in sts:
                st.wait()
            gather_desc(g + NBUF, b).start()
        return carry

    lax.fori_loop(0, N_OUTER - 1, body, 0)

    last = []
    for b in range(NBUF):
        g = N_CHUNKS - NBUF + b
        gather_desc(g, b).wait()
        sts = store_descs(g, b)
        for st in sts:
            st.start()
        last.extend(sts)
    for st in last:
        st.wait()


def kernel(phoneme_ids, embedding_table):
    idx = phoneme_ids.reshape(-1).astype(jnp.int32)
    return _gather_kernel(embedding_table, idx)


# speedup vs baseline: 2.1212x; 1.1717x over previous
import functools

import jax
import jax.numpy as jnp
from jax import lax
from jax.experimental import pallas as pl
from jax.experimental.pallas import tpu as pltpu
from jax.experimental.pallas import tpu_sc as plsc

NUM_ROWS = 1_000_000
DIM = 32
IDS = 16384
SEQ = 50
BATCH = IDS * SEQ

NC = 2
NS = 16
NW = NC * NS
IDS_PER_W = IDS // NW
B_PER_W = BATCH // NW
TBLK = 16
CHUNK = TBLK * SEQ
N_BLKS = IDS_PER_W // TBLK
NBUF = 2

_mesh = plsc.VectorSubcoreMesh(core_axis_name="c", subcore_axis_name="s")


@functools.partial(
    pl.kernel,
    mesh=_mesh,
    out_type=jax.ShapeDtypeStruct((SEQ * DIM, IDS), jnp.float32),
    compiler_params=pltpu.CompilerParams(
        use_tc_tiling_on_sc=False, needs_layout_passes=False
    ),
    scratch_types=[
        pltpu.VMEM((B_PER_W,), jnp.int32),
        [pltpu.VMEM((CHUNK, DIM), jnp.float32) for _ in range(NBUF)],
        [pltpu.VMEM((SEQ * DIM, TBLK), jnp.float32) for _ in range(NBUF)],
        [pltpu.SemaphoreType.DMA for _ in range(NBUF)],
        [pltpu.SemaphoreType.DMA for _ in range(NBUF)],
    ],
)
def _gather_kernel(table_hbm, idx_hbm, out_hbm, idx_v, rows, trans, sem_g,
                   sem_o):
    wid = lax.axis_index("s") * NC + lax.axis_index("c")
    base = wid * B_PER_W
    id0 = wid * IDS_PER_W

    pltpu.sync_copy(idx_hbm.at[pl.ds(base, B_PER_W)], idx_v)

    lane = lax.iota(jnp.int32, 16)

    def gather_desc(g, b):
        return pltpu.make_async_copy(
            table_hbm.at[idx_v.at[pl.ds(g * CHUNK, CHUNK)]], rows[b], sem_g[b]
        )

    def store_descs(g, b):
        return [
            pltpu.make_async_copy(
                trans[b],
                out_hbm.at[:, pl.ds(id0 + g * TBLK, TBLK)],
                sem_o[b],
            )
        ]

    def transpose_block(b):
        t_vecs = [jnp.full((16,), t, jnp.int32) for t in range(TBLK)]

        def j_body(j, carry):
            row_lo = lane + j * jnp.int32(DIM)
            row_hi = row_lo + jnp.int32(16)
            for t in range(TBLK):
                r = t * SEQ + j
                lo = rows[b][r, pl.ds(0, 16)]
                hi = rows[b][r, pl.ds(16, 16)]
                plsc.store_scatter(trans[b], [row_lo, t_vecs[t]], lo)
                plsc.store_scatter(trans[b], [row_hi, t_vecs[t]], hi)
            return carry

        lax.fori_loop(0, SEQ, j_body, 0)

    for b in range(NBUF):
        gather_desc(b, b).start()

    def body(outer, carry):
        for b in range(NBUF):
            g = outer * NBUF + b

            def wait_prev(g=g, b=b):
                for st in store_descs(g - NBUF, b):
                    st.wait()

            pl.when(g >= NBUF)(wait_prev)
            gather_desc(g, b).wait()
            transpose_block(b)
            gather_desc(g + NBUF, b).start()
            for st in store_descs(g, b):
                st.start()
        return carry

    lax.fori_loop(0, N_BLKS // NBUF - 1, body, 0)

    for b in range(NBUF):
        g = N_BLKS - NBUF + b
        for st in store_descs(g - NBUF, b):
            st.wait()
        gather_desc(g, b).wait()
        transpose_block(b)
        for st in store_descs(g, b):
            st.start()
    for b in range(NBUF):
        g = N_BLKS - NBUF + b
        for st in store_descs(g, b):
            st.wait()


def kernel(phoneme_ids, embedding_table):
    idx = phoneme_ids.reshape(-1).astype(jnp.int32)
    out_t = _gather_kernel(embedding_table, idx)
    return out_t.reshape(SEQ, DIM, IDS).transpose(2, 0, 1)
